# baseline (speedup 1.0000x reference)
import jax
import jax.numpy as jnp
from jax import lax
from jax.experimental import pallas as pl
from jax.experimental.pallas import tpu as pltpu
from jax.experimental.pallas import tpu_sc as plsc

N_NODES = 10000
N_PAIRS = 320000
D_FEAT = 128
HIDDEN = 128
N_CLASSES = 64
NUM_PPR = 3


_PROWS = N_PAIRS // 2
_BP = 1280
_GROUPS = _PROWS // 128


def _mlp_body(x_ref, se0_ref, so0_ref, se1_ref, so1_ref, se2_ref, so2_ref,
              wsq_ref, w0_ref, w1_ref, w2_ref, wh_ref,
              o0_ref, o1_ref, o2_ref):
    f32 = jnp.float32
    hi = jax.lax.Precision.HIGHEST
    a = jnp.maximum(jnp.dot(x_ref[...], w0_ref[...], preferred_element_type=f32, precision=hi), 0.0)
    b = jnp.maximum(jnp.dot(a, w1_ref[...], preferred_element_type=f32, precision=hi), 0.0)
    l = jnp.dot(b, w2_ref[...], preferred_element_type=f32, precision=hi)
    h = jnp.dot(l, wh_ref[...], preferred_element_type=f32, precision=hi)

    def scale(se_ref, so_ref, w):
        return jnp.concatenate(
            [jnp.broadcast_to(se_ref[...], (_BP, N_CLASSES)),
             jnp.broadcast_to(so_ref[...], (_BP, N_CLASSES))], axis=1) * w
    o0_ref[...] = h * scale(se0_ref, so0_ref, wsq_ref[0, 0])
    o1_ref[...] = h * scale(se1_ref, so1_ref, wsq_ref[1, 0])
    o2_ref[...] = h * scale(se2_ref, so2_ref, wsq_ref[2, 0])


def _mlp_scaled(X, s0, s1, s2, W_squeeze, W0, W1, W2, W_head):
    grid = (_PROWS // _BP,)
    X2 = X.reshape(_PROWS, 2 * D_FEAT)
    z = jnp.zeros((D_FEAT, HIDDEN), jnp.float32)
    bd = lambda W: jnp.block([[W, z], [z, W]])
    W0_2, W1_2, W2_2 = bd(W0), bd(W1), bd(W2)
    Wh_2 = jnp.concatenate(
        [jnp.concatenate([W_head, jnp.zeros((HIDDEN, N_CLASSES), jnp.float32)], axis=1),
         jnp.concatenate([jnp.zeros((HIDDEN, N_CLASSES), jnp.float32), W_head], axis=1)],
        axis=0)
    row_spec = pl.BlockSpec((_BP, 2 * D_FEAT), lambda i: (i, 0))
    sc_spec = pl.BlockSpec((_BP, 1), lambda i: (i, 0))
    out_spec = pl.BlockSpec((_BP, 2 * N_CLASSES), lambda i: (i, 0))
    w_spec = lambda r, c: pl.BlockSpec((r, c), lambda i: (0, 0))
    out_sds = jax.ShapeDtypeStruct((_PROWS, 2 * N_CLASSES), jnp.float32)
    eo = lambda sc: (sc.reshape(_PROWS, 2)[:, :1], sc.reshape(_PROWS, 2)[:, 1:])
    se0, so0 = eo(s0)
    se1, so1 = eo(s1)
    se2, so2 = eo(s2)
    return pl.pallas_call(
        _mlp_body,
        grid=grid,
        in_specs=[
            row_spec, sc_spec, sc_spec, sc_spec, sc_spec, sc_spec, sc_spec,
            pl.BlockSpec(memory_space=pltpu.SMEM),
            w_spec(2 * D_FEAT, 2 * HIDDEN), w_spec(2 * HIDDEN, 2 * HIDDEN),
            w_spec(2 * HIDDEN, 2 * HIDDEN), w_spec(2 * HIDDEN, 2 * N_CLASSES),
        ],
        out_specs=[out_spec, out_spec, out_spec],
        out_shape=[out_sds, out_sds, out_sds],
    )(X2, se0, so0, se1, so1, se2, so2, W_squeeze, W0_2, W1_2, W2_2, Wh_2)


_ACC_ROWS = 10112
_DUMMY = N_NODES
_NT = 16
_BASE_G = _GROUPS // _NT
_EXTRA_G = _GROUPS - _BASE_G * _NT


def _sc_scatter_body(s0, s1, s2, ie0, ie1, ie2, io0, io1, io2, out,
                     sbuf, ibuf, acc):
    c = lax.axis_index("c")
    s = lax.axis_index("s")

    def _zrow(r, carry):
        for v in range(8):
            sbuf[r, pl.ds(v * 16, 16)] = jnp.zeros((16,), jnp.float32)
        return carry
    lax.fori_loop(0, 128, _zrow, 0)
    for g_off in range(5):
        g = s + g_off * 16
        @pl.when(g < _ACC_ROWS // 128)
        def _():
            pltpu.sync_copy(sbuf, acc.at[pl.ds(g * 128, 128)])
    plsc.subcore_barrier()

    start_g = s * _BASE_G + jnp.minimum(s, _EXTRA_G)
    n = _BASE_G + (s < _EXTRA_G).astype(jnp.int32)

    for s_hbm, ie_hbm, io_hbm in ((s0, ie0, io0), (s1, ie1, io1), (s2, ie2, io2)):
        def _grp(j, carry, s_hbm=s_hbm, ie_hbm=ie_hbm, io_hbm=io_hbm):
            row = start_g + j
            @pl.when(c == 0)
            def _():
                pltpu.sync_copy(ie_hbm.at[row, 0], ibuf)
            @pl.when(c == 1)
            def _():
                pltpu.sync_copy(io_hbm.at[row, 0], ibuf)
            pltpu.sync_copy(s_hbm.at[pl.ds(row * 128, 128)], sbuf)
            pltpu.sync_copy(sbuf, acc.at[ibuf], add=True)
            return carry
        lax.fori_loop(0, n, _grp, 0)

    plsc.subcore_barrier()
    w_start = s * 624
    pltpu.sync_copy(acc.at[pl.ds(w_start, 624)],
                    out.at[c].at[pl.ds(w_start, 624)])
    @pl.when(s == 15)
    def _():
        pltpu.sync_copy(acc.at[pl.ds(9984, 16)],
                        out.at[c].at[pl.ds(9984, 16)])


def _sc_scatter(S0, S1, S2, idx0, idx1, idx2):
    mesh = plsc.VectorSubcoreMesh(core_axis_name="c", subcore_axis_name="s")
    kfn = pl.kernel(
        _sc_scatter_body,
        out_type=jax.ShapeDtypeStruct((2, N_NODES, HIDDEN), jnp.float32),
        mesh=mesh,
        scratch_types=[
            pltpu.VMEM((128, HIDDEN), jnp.float32),
            pltpu.VMEM((128,), jnp.int32),
            pltpu.VMEM_SHARED((_ACC_ROWS, HIDDEN), jnp.float32),
        ],
    )
    def _eo(i):
        i2 = i.reshape(_PROWS, 2).astype(jnp.int32)
        return (i2[:, 0].reshape(_GROUPS, 1, 128),
                i2[:, 1].reshape(_GROUPS, 1, 128))
    e0, o0 = _eo(idx0)
    e1, o1 = _eo(idx1)
    e2, o2 = _eo(idx2)
    return kfn(S0, S1, S2, e0, e1, e2, o0, o1, o2)


def _combine_body(p_ref, wh_ref, bsq_ref, bh_ref, o_ref):
    bias = bsq_ref[0, 0] * jnp.sum(wh_ref[...], axis=0, keepdims=True) + bh_ref[...]
    o_ref[...] = p_ref[0][:, :N_CLASSES] + p_ref[1][:, N_CLASSES:] + bias


def _combine(partials, W_head, b_squeeze, b_head):
    blk = 2000
    return pl.pallas_call(
        _combine_body,
        grid=(N_NODES // blk,),
        in_specs=[
            pl.BlockSpec((2, blk, HIDDEN), lambda i: (0, i, 0)),
            pl.BlockSpec((HIDDEN, N_CLASSES), lambda i: (0, 0)),
            pl.BlockSpec(memory_space=pltpu.SMEM),
            pl.BlockSpec((1, N_CLASSES), lambda i: (0, 0)),
        ],
        out_specs=pl.BlockSpec((blk, N_CLASSES), lambda i: (i, 0)),
        out_shape=jax.ShapeDtypeStruct((N_NODES, N_CLASSES), jnp.float32),
    )(partials, W_head, b_squeeze[:, None], b_head[None, :])


def kernel(X, ppr_scores_0, ppr_scores_1, ppr_scores_2,
           ppr_idx_0, ppr_idx_1, ppr_idx_2,
           W0, W1, W2, W_squeeze, b_squeeze, W_head, b_head):
    S0, S1, S2 = _mlp_scaled(X, ppr_scores_0, ppr_scores_1, ppr_scores_2,
                             W_squeeze, W0, W1, W2, W_head)
    partials = _sc_scatter(S0, S1, S2, ppr_idx_0, ppr_idx_1, ppr_idx_2)
    return _combine(partials, W_head, b_squeeze, b_head)


# speedup vs baseline: 1.0365x; 1.0365x over previous
import jax
import jax.numpy as jnp
from jax import lax
from jax.experimental import pallas as pl
from jax.experimental.pallas import tpu as pltpu
from jax.experimental.pallas import tpu_sc as plsc

N_NODES = 10000
N_PAIRS = 320000
D_FEAT = 128
HIDDEN = 128
N_CLASSES = 64
NUM_PPR = 3


_PROWS = N_PAIRS // 2
_BP = 1280
_GROUPS = _PROWS // 128


def _mlp_body(x_ref, se0_ref, so0_ref, se1_ref, so1_ref, se2_ref, so2_ref,
              wsq_ref, w0_ref, w1_ref, w2_ref, wh_ref,
              o0_ref, o1_ref, o2_ref):
    f32 = jnp.float32
    hi = jax.lax.Precision.HIGHEST
    a = jnp.maximum(jnp.dot(x_ref[...], w0_ref[...], preferred_element_type=f32, precision=hi), 0.0)
    b = jnp.maximum(jnp.dot(a, w1_ref[...], preferred_element_type=f32, precision=hi), 0.0)
    l = jnp.dot(b, w2_ref[...], preferred_element_type=f32, precision=hi)
    h = jnp.dot(l, wh_ref[...], preferred_element_type=f32, precision=hi)

    g = _BP // 128
    h3 = h.reshape(g, 128, 2 * N_CLASSES)
    lane = jax.lax.broadcasted_iota(jnp.int32, (g, 128, 2 * N_CLASSES), 2)

    def scaled(se_ref, so_ref, w):
        lo = h3 * (se_ref[0][..., None] * w)
        hi_ = h3 * (so_ref[0][..., None] * w)
        return jnp.where(lane < N_CLASSES, lo, hi_).reshape(_BP, 2 * N_CLASSES)
    o0_ref[...] = scaled(se0_ref, so0_ref, wsq_ref[0, 0])
    o1_ref[...] = scaled(se1_ref, so1_ref, wsq_ref[1, 0])
    o2_ref[...] = scaled(se2_ref, so2_ref, wsq_ref[2, 0])


def _mlp_scaled(X, s0, s1, s2, W_squeeze, W0, W1, W2, W_head):
    grid = (_PROWS // _BP,)
    X2 = X.reshape(_PROWS, 2 * D_FEAT)
    z = jnp.zeros((D_FEAT, HIDDEN), jnp.float32)
    bd = lambda W: jnp.block([[W, z], [z, W]])
    W0_2, W1_2, W2_2 = bd(W0), bd(W1), bd(W2)
    zh = jnp.zeros((HIDDEN, N_CLASSES), jnp.float32)
    Wh_2 = jnp.block([[W_head, zh], [zh, W_head]])
    gg = _BP // 128
    row_spec = pl.BlockSpec((_BP, 2 * D_FEAT), lambda i: (i, 0))
    sc_spec = pl.BlockSpec((1, gg, 128), lambda i: (i, 0, 0))
    out_spec = pl.BlockSpec((_BP, 2 * N_CLASSES), lambda i: (i, 0))
    w_spec = lambda r, c: pl.BlockSpec((r, c), lambda i: (0, 0))
    out_sds = jax.ShapeDtypeStruct((_PROWS, 2 * N_CLASSES), jnp.float32)
    nst = _PROWS // _BP
    eo = lambda sc: (sc.reshape(_PROWS, 2)[:, 0].reshape(nst, gg, 128),
                     sc.reshape(_PROWS, 2)[:, 1].reshape(nst, gg, 128))
    se0, so0 = eo(s0)
    se1, so1 = eo(s1)
    se2, so2 = eo(s2)
    return pl.pallas_call(
        _mlp_body,
        grid=grid,
        in_specs=[
            row_spec, sc_spec, sc_spec, sc_spec, sc_spec, sc_spec, sc_spec,
            pl.BlockSpec(memory_space=pltpu.SMEM),
            w_spec(2 * D_FEAT, 2 * HIDDEN), w_spec(2 * HIDDEN, 2 * HIDDEN),
            w_spec(2 * HIDDEN, 2 * HIDDEN), w_spec(2 * HIDDEN, 2 * N_CLASSES),
        ],
        out_specs=[out_spec, out_spec, out_spec],
        out_shape=[out_sds, out_sds, out_sds],
    )(X2, se0, so0, se1, so1, se2, so2, W_squeeze, W0_2, W1_2, W2_2, Wh_2)


_ACC_ROWS = 10112
_DUMMY = N_NODES
_NT = 16
_BASE_G = _GROUPS // _NT
_EXTRA_G = _GROUPS - _BASE_G * _NT


def _sc_scatter_body(s0, s1, s2, ie0, ie1, ie2, io0, io1, io2, out,
                     sbuf, ibuf, acc):
    c = lax.axis_index("c")
    s = lax.axis_index("s")

    def _zrow(r, carry):
        for v in range(8):
            sbuf[r, pl.ds(v * 16, 16)] = jnp.zeros((16,), jnp.float32)
        return carry
    lax.fori_loop(0, 128, _zrow, 0)
    for g_off in range(5):
        g = s + g_off * 16
        @pl.when(g < _ACC_ROWS // 128)
        def _():
            pltpu.sync_copy(sbuf, acc.at[pl.ds(g * 128, 128)])
    plsc.subcore_barrier()

    start_g = s * _BASE_G + jnp.minimum(s, _EXTRA_G)
    n = _BASE_G + (s < _EXTRA_G).astype(jnp.int32)

    for s_hbm, ie_hbm, io_hbm in ((s0, ie0, io0), (s1, ie1, io1), (s2, ie2, io2)):
        def _grp(j, carry, s_hbm=s_hbm, ie_hbm=ie_hbm, io_hbm=io_hbm):
            row = start_g + j
            @pl.when(c == 0)
            def _():
                pltpu.sync_copy(ie_hbm.at[row, 0], ibuf)
            @pl.when(c == 1)
            def _():
                pltpu.sync_copy(io_hbm.at[row, 0], ibuf)
            pltpu.sync_copy(s_hbm.at[pl.ds(row * 128, 128)], sbuf)
            pltpu.sync_copy(sbuf, acc.at[ibuf], add=True)
            return carry
        lax.fori_loop(0, n, _grp, 0)

    plsc.subcore_barrier()
    w_start = s * 624
    pltpu.sync_copy(acc.at[pl.ds(w_start, 624)],
                    out.at[c].at[pl.ds(w_start, 624)])
    @pl.when(s == 15)
    def _():
        pltpu.sync_copy(acc.at[pl.ds(9984, 16)],
                        out.at[c].at[pl.ds(9984, 16)])


def _sc_scatter(S0, S1, S2, idx0, idx1, idx2):
    mesh = plsc.VectorSubcoreMesh(core_axis_name="c", subcore_axis_name="s")
    kfn = pl.kernel(
        _sc_scatter_body,
        out_type=jax.ShapeDtypeStruct((2, N_NODES, HIDDEN), jnp.float32),
        mesh=mesh,
        scratch_types=[
            pltpu.VMEM((128, HIDDEN), jnp.float32),
            pltpu.VMEM((128,), jnp.int32),
            pltpu.VMEM_SHARED((_ACC_ROWS, HIDDEN), jnp.float32),
        ],
    )
    def _eo(i):
        i2 = i.reshape(_PROWS, 2).astype(jnp.int32)
        return (i2[:, 0].reshape(_GROUPS, 1, 128),
                i2[:, 1].reshape(_GROUPS, 1, 128))
    e0, o0 = _eo(idx0)
    e1, o1 = _eo(idx1)
    e2, o2 = _eo(idx2)
    return kfn(S0, S1, S2, e0, e1, e2, o0, o1, o2)


def _combine_body(p_ref, wh_ref, bsq_ref, bh_ref, o_ref):
    bias = bsq_ref[0, 0] * jnp.sum(wh_ref[...], axis=0, keepdims=True) + bh_ref[...]
    o_ref[...] = p_ref[0][:, :N_CLASSES] + p_ref[1][:, N_CLASSES:] + bias


def _combine(partials, W_head, b_squeeze, b_head):
    blk = 2000
    return pl.pallas_call(
        _combine_body,
        grid=(N_NODES // blk,),
        in_specs=[
            pl.BlockSpec((2, blk, HIDDEN), lambda i: (0, i, 0)),
            pl.BlockSpec((HIDDEN, N_CLASSES), lambda i: (0, 0)),
            pl.BlockSpec(memory_space=pltpu.SMEM),
            pl.BlockSpec((1, N_CLASSES), lambda i: (0, 0)),
        ],
        out_specs=pl.BlockSpec((blk, N_CLASSES), lambda i: (i, 0)),
        out_shape=jax.ShapeDtypeStruct((N_NODES, N_CLASSES), jnp.float32),
    )(partials, W_head, b_squeeze[:, None], b_head[None, :])


def kernel(X, ppr_scores_0, ppr_scores_1, ppr_scores_2,
           ppr_idx_0, ppr_idx_1, ppr_idx_2,
           W0, W1, W2, W_squeeze, b_squeeze, W_head, b_head):
    S0, S1, S2 = _mlp_scaled(X, ppr_scores_0, ppr_scores_1, ppr_scores_2,
                             W_squeeze, W0, W1, W2, W_head)
    partials = _sc_scatter(S0, S1, S2, ppr_idx_0, ppr_idx_1, ppr_idx_2)
    return _combine(partials, W_head, b_squeeze, b_head)


# speedup vs baseline: 1.7910x; 1.7279x over previous
import jax
import jax.numpy as jnp
from jax import lax
from jax.experimental import pallas as pl
from jax.experimental.pallas import tpu as pltpu
from jax.experimental.pallas import tpu_sc as plsc

N_NODES = 10000
N_PAIRS = 320000
D_FEAT = 128
HIDDEN = 128
N_CLASSES = 64
NUM_PPR = 3


_BR = 2560
_GROUPS = N_PAIRS // 128


def _mlp_body(x_ref, s0_ref, s1_ref, s2_ref, wsq_ref,
              w0_ref, w1_ref, w2_ref,
              o0_ref, o1_ref, o2_ref):
    f32 = jnp.float32
    a = jnp.maximum(jnp.dot(x_ref[...], w0_ref[...], preferred_element_type=f32, precision=jax.lax.Precision.HIGHEST), 0.0)
    b = jnp.maximum(jnp.dot(a, w1_ref[...], preferred_element_type=f32, precision=jax.lax.Precision.HIGHEST), 0.0)
    l = jnp.dot(b, w2_ref[...], preferred_element_type=f32, precision=jax.lax.Precision.HIGHEST)
    o0_ref[...] = l * (s0_ref[...] * wsq_ref[0, 0])
    o1_ref[...] = l * (s1_ref[...] * wsq_ref[1, 0])
    o2_ref[...] = l * (s2_ref[...] * wsq_ref[2, 0])


def _mlp_scaled(X, s0, s1, s2, W_squeeze, W0, W1, W2):
    grid = (N_PAIRS // _BR,)
    row_spec = pl.BlockSpec((_BR, D_FEAT), lambda i: (i, 0))
    sc_spec = pl.BlockSpec((_BR, 1), lambda i: (i, 0))
    out_spec = pl.BlockSpec((_BR, HIDDEN), lambda i: (i, 0))
    w_spec = lambda r, c: pl.BlockSpec((r, c), lambda i: (0, 0))
    out_sds = jax.ShapeDtypeStruct((N_PAIRS, HIDDEN), jnp.float32)
    return pl.pallas_call(
        _mlp_body,
        grid=grid,
        in_specs=[
            row_spec, sc_spec, sc_spec, sc_spec,
            pl.BlockSpec(memory_space=pltpu.SMEM),
            w_spec(D_FEAT, HIDDEN), w_spec(HIDDEN, HIDDEN),
            w_spec(HIDDEN, HIDDEN),
        ],
        out_specs=[out_spec, out_spec, out_spec],
        out_shape=[out_sds, out_sds, out_sds],
    )(X, s0[:, None], s1[:, None], s2[:, None], W_squeeze, W0, W1, W2)


_ACC_ROWS = 10112
_NW = 32
_BASE_G = _GROUPS // _NW
_EXTRA_G = _GROUPS - _BASE_G * _NW


def _sc_scatter_body(s0, s1, s2, i0, i1, i2, out,
                     sbuf, ibuf, zbuf, acc):
    c = lax.axis_index("c")
    s = lax.axis_index("s")
    wid = s * 2 + c

    def _zrow(r, carry):
        for v in range(8):
            zbuf[r, pl.ds(v * 16, 16)] = jnp.zeros((16,), jnp.float32)
        return carry
    lax.fori_loop(0, 128, _zrow, 0)
    for g_off in range(5):
        g = s + g_off * 16
        @pl.when(g < _ACC_ROWS // 128)
        def _():
            pltpu.sync_copy(zbuf, acc.at[pl.ds(g * 128, 128)])
    plsc.subcore_barrier()

    start = wid * _BASE_G + jnp.minimum(wid, _EXTRA_G)
    n = _BASE_G + (wid < _EXTRA_G).astype(jnp.int32)

    for s_hbm, i_hbm in ((s0, i0), (s1, i1), (s2, i2)):
        def _grp(j, carry, s_hbm=s_hbm, i_hbm=i_hbm):
            row = start + j
            pltpu.sync_copy(i_hbm.at[row, 0], ibuf)
            pltpu.sync_copy(s_hbm.at[pl.ds(row * 128, 128)], sbuf)
            pltpu.sync_copy(sbuf, acc.at[ibuf], add=True)
            return carry
        lax.fori_loop(0, n, _grp, 0)

    plsc.subcore_barrier()
    w_start = s * 624
    pltpu.sync_copy(acc.at[pl.ds(w_start, 624)],
                    out.at[c].at[pl.ds(w_start, 624)])
    @pl.when(s == 15)
    def _():
        pltpu.sync_copy(acc.at[pl.ds(9984, 16)],
                        out.at[c].at[pl.ds(9984, 16)])


def _sc_scatter(S0, S1, S2, idx0, idx1, idx2):
    mesh = plsc.VectorSubcoreMesh(core_axis_name="c", subcore_axis_name="s")
    kfn = pl.kernel(
        _sc_scatter_body,
        out_type=jax.ShapeDtypeStruct((2, N_NODES, HIDDEN), jnp.float32),
        mesh=mesh,
        scratch_types=[
            pltpu.VMEM((128, HIDDEN), jnp.float32),
            pltpu.VMEM((128,), jnp.int32),
            pltpu.VMEM((128, HIDDEN), jnp.float32),
            pltpu.VMEM_SHARED((_ACC_ROWS, HIDDEN), jnp.float32),
        ],
    )
    i3d = lambda i: i.reshape(_GROUPS, 1, 128).astype(jnp.int32)
    return kfn(S0, S1, S2, i3d(idx0), i3d(idx1), i3d(idx2))


def _combine_body(p_ref, wh_ref, bsq_ref, bh_ref, o_ref):
    x2d = p_ref[0] + p_ref[1] + bsq_ref[0, 0]
    o_ref[...] = jnp.dot(x2d, wh_ref[...], preferred_element_type=jnp.float32,
                         precision=jax.lax.Precision.HIGHEST) + bh_ref[...]


def _combine(partials, W_head, b_squeeze, b_head):
    blk = 2000
    return pl.pallas_call(
        _combine_body,
        grid=(N_NODES // blk,),
        in_specs=[
            pl.BlockSpec((2, blk, HIDDEN), lambda i: (0, i, 0)),
            pl.BlockSpec((HIDDEN, N_CLASSES), lambda i: (0, 0)),
            pl.BlockSpec(memory_space=pltpu.SMEM),
            pl.BlockSpec((1, N_CLASSES), lambda i: (0, 0)),
        ],
        out_specs=pl.BlockSpec((blk, N_CLASSES), lambda i: (i, 0)),
        out_shape=jax.ShapeDtypeStruct((N_NODES, N_CLASSES), jnp.float32),
    )(partials, W_head, b_squeeze[:, None], b_head[None, :])


def kernel(X, ppr_scores_0, ppr_scores_1, ppr_scores_2,
           ppr_idx_0, ppr_idx_1, ppr_idx_2,
           W0, W1, W2, W_squeeze, b_squeeze, W_head, b_head):
    S0, S1, S2 = _mlp_scaled(X, ppr_scores_0, ppr_scores_1, ppr_scores_2,
                             W_squeeze, W0, W1, W2)
    partials = _sc_scatter(S0, S1, S2, ppr_idx_0, ppr_idx_1, ppr_idx_2)
    return _combine(partials, W_head, b_squeeze, b_head)
